# baseline (speedup 1.0000x reference)
import functools

import jax
import jax.numpy as jnp
from jax import lax
from jax.experimental import pallas as pl
from jax.experimental.pallas import tpu as pltpu
from jax.experimental.pallas import tpu_sc as plsc

import numpy as np

_N_CLUSTERS = 512
_DIM = 32
_TOK_BLOCK = 4096
_IOTA_COL = np.arange(_N_CLUSTERS, dtype=np.float32)[:, None]


def _assign_body(x_ref, c_ref, idx_ref):
    x = x_ref[...]
    c = c_ref[...]
    norm = jnp.sqrt(jnp.sum(x * x, axis=1, keepdims=True))
    xn = x / jnp.maximum(norm, 1e-12)
    cn = jnp.sum(c * c, axis=1)[:, None]
    scores = cn + lax.dot_general(
        c * -2.0, xn, (((1,), (1,)), ((), ())),
        preferred_element_type=jnp.float32,
        precision=lax.Precision.DEFAULT)
    m = jnp.min(scores, axis=0)
    iota_col = lax.broadcasted_iota(
        jnp.int32, (_N_CLUSTERS, 1), 0).astype(jnp.float32)
    cand = jnp.where(scores == m[None, :], iota_col, float(_N_CLUSTERS))
    idx = jnp.min(cand, axis=0).astype(jnp.int32)
    idx_ref[...] = idx.reshape(_TOK_BLOCK // 128, 128)


def _gather_body_loop(info, per_worker, chunk, n_chunks, groups, bufs,
                      table_hbm, idx_hbm, out_hbm, table_v):
    wid = lax.axis_index("s") * info.num_cores + lax.axis_index("c")
    base = wid * per_worker
    pltpu.sync_copy(table_hbm, table_v)
    lane = lax.iota(jnp.int32, 16)

    out_copies = []
    for k in range(n_chunks):
        idx_v, rows_v, sem = bufs[k % len(bufs)]
        off = base + k * chunk
        pltpu.sync_copy(idx_hbm.at[pl.ds(off // 128, chunk // 128)], idx_v)

        @plsc.parallel_loop(0, groups, unroll=2)
        def body(g):
            tok = g * 16 + lane
            row_idx = idx_v[g // 8, pl.ds((g % 8) * 16, 16)]
            for r in range(_DIM):
                rot = jnp.bitwise_and(lane + r, _DIM - 1)
                vals = plsc.load_gather(table_v, [rot, row_idx])
                plsc.store_scatter(rows_v, [tok, rot], vals)

        out_copies.append(
            pltpu.async_copy(rows_v, out_hbm.at[pl.ds(off, chunk)], sem))
    for c in out_copies:
        c.wait()


def _assign(X, centroids, start_blk, n_blk):
    return pl.pallas_call(
        _assign_body,
        grid=(n_blk,),
        in_specs=[
            pl.BlockSpec((_TOK_BLOCK, _DIM), lambda i: (i + start_blk, 0)),
            pl.BlockSpec((_N_CLUSTERS, _DIM), lambda i: (0, 0)),
        ],
        out_specs=pl.BlockSpec((_TOK_BLOCK // 128, 128),
                               lambda i: (i, 0)),
        out_shape=jax.ShapeDtypeStruct((n_blk * _TOK_BLOCK // 128, 128),
                                       jnp.int32),
    )(X, centroids)


@functools.cache
def _make_gather(n_tokens):
    info = plsc.get_sparse_core_info()
    num_workers = info.num_cores * info.num_subcores
    per_worker = n_tokens // num_workers
    chunk = per_worker
    while chunk > 1024 and chunk % 256 == 0:
        chunk //= 2
    assert chunk % 128 == 0 and per_worker % chunk == 0
    n_chunks = per_worker // chunk
    groups = chunk // 16
    mesh = plsc.VectorSubcoreMesh(core_axis_name="c", subcore_axis_name="s")

    @functools.partial(
        pl.kernel,
        out_type=jax.ShapeDtypeStruct((n_tokens, _DIM), jnp.float32),
        mesh=mesh,
        scratch_types=[
            pltpu.VMEM((_DIM, _N_CLUSTERS), jnp.float32),
            pltpu.VMEM((chunk // 128, 128), jnp.int32),
            pltpu.VMEM((chunk, _DIM), jnp.float32),
            pltpu.SemaphoreType.DMA,
            pltpu.VMEM((chunk // 128, 128), jnp.int32),
            pltpu.VMEM((chunk, _DIM), jnp.float32),
            pltpu.SemaphoreType.DMA,
        ],
        compiler_params=pltpu.CompilerParams(
            use_tc_tiling_on_sc=False, needs_layout_passes=False),
    )
    def gather(table_hbm, idx_hbm, out_hbm, table_v,
               idx_a, rows_a, sem_a, idx_b, rows_b, sem_b):
        bufs = [(idx_a, rows_a, sem_a), (idx_b, rows_b, sem_b)]
        _gather_body_loop(info, per_worker, chunk, n_chunks, groups, bufs,
                          table_hbm, idx_hbm, out_hbm, table_v)

    return gather


def kernel(X, centroids, ema_cluster_size, ema_w):
    n = X.shape[0]
    n_blk = n // _TOK_BLOCK
    parts_blk = [n_blk // 2, n_blk - n_blk // 2]
    table_t = centroids.T
    idxs, qs = [], []
    start = 0
    for pb in parts_blk:
        idx_p = _assign(X, centroids, start, pb)
        idxs.append(idx_p)
        qs.append(_make_gather(pb * _TOK_BLOCK)(table_t, idx_p))
        start += pb
    quantized = jnp.concatenate(qs, axis=0)
    idx = jnp.concatenate(idxs, axis=0).reshape(-1)[:, None]
    return (quantized, idx)


# speedup vs baseline: 1.0817x; 1.0817x over previous
import functools

import jax
import jax.numpy as jnp
from jax import lax
from jax.experimental import pallas as pl
from jax.experimental.pallas import tpu as pltpu
from jax.experimental.pallas import tpu_sc as plsc


_N_CLUSTERS = 512
_DIM = 32
_TOK_BLOCK = 4096


def _assign_body(x_ref, c_ref, idx_ref):
    x = x_ref[...]
    c = c_ref[...]
    norm = jnp.sqrt(jnp.sum(x * x, axis=1, keepdims=True))
    xn = x / jnp.maximum(norm, 1e-12)
    cn = jnp.sum(c * c, axis=1)[:, None]
    scores = cn + lax.dot_general(
        c * -2.0, xn, (((1,), (1,)), ((), ())),
        preferred_element_type=jnp.float32,
        precision=lax.Precision.DEFAULT)
    idx = jnp.argmin(scores, axis=0).astype(jnp.int32)
    idx_ref[...] = idx.reshape(_TOK_BLOCK // 128, 128)


def _gather_body_loop(info, per_worker, chunk, n_chunks, groups, bufs,
                      table_hbm, idx_hbm, out_hbm, table_v):
    wid = lax.axis_index("s") * info.num_cores + lax.axis_index("c")
    base = wid * per_worker
    pltpu.sync_copy(table_hbm, table_v)
    lane = lax.iota(jnp.int32, 16)

    out_copies = []
    for k in range(n_chunks):
        idx_v, rows_v, sem = bufs[k % len(bufs)]
        off = base + k * chunk
        pltpu.sync_copy(idx_hbm.at[pl.ds(off // 128, chunk // 128)], idx_v)

        @plsc.parallel_loop(0, groups, unroll=2)
        def body(g):
            tok = g * 16 + lane
            row_idx = idx_v[g // 8, pl.ds((g % 8) * 16, 16)]
            for r in range(_DIM):
                rot = jnp.bitwise_and(lane + r, _DIM - 1)
                vals = plsc.load_gather(table_v, [rot, row_idx])
                plsc.store_scatter(rows_v, [tok, rot], vals)

        out_copies.append(
            pltpu.async_copy(rows_v, out_hbm.at[pl.ds(off, chunk)], sem))
    for c in out_copies:
        c.wait()


def _assign(X, centroids, start_blk, n_blk):
    return pl.pallas_call(
        _assign_body,
        grid=(n_blk,),
        in_specs=[
            pl.BlockSpec((_TOK_BLOCK, _DIM), lambda i: (i + start_blk, 0)),
            pl.BlockSpec((_N_CLUSTERS, _DIM), lambda i: (0, 0)),
        ],
        out_specs=pl.BlockSpec((_TOK_BLOCK // 128, 128),
                               lambda i: (i, 0)),
        out_shape=jax.ShapeDtypeStruct((n_blk * _TOK_BLOCK // 128, 128),
                                       jnp.int32),
    )(X, centroids)


@functools.cache
def _make_gather(n_tokens):
    info = plsc.get_sparse_core_info()
    num_workers = info.num_cores * info.num_subcores
    per_worker = n_tokens // num_workers
    chunk = per_worker
    while chunk > 1024 and chunk % 256 == 0:
        chunk //= 2
    assert chunk % 128 == 0 and per_worker % chunk == 0
    n_chunks = per_worker // chunk
    groups = chunk // 16
    mesh = plsc.VectorSubcoreMesh(core_axis_name="c", subcore_axis_name="s")

    @functools.partial(
        pl.kernel,
        out_type=jax.ShapeDtypeStruct((n_tokens, _DIM), jnp.float32),
        mesh=mesh,
        scratch_types=[
            pltpu.VMEM((_DIM, _N_CLUSTERS), jnp.float32),
            pltpu.VMEM((chunk // 128, 128), jnp.int32),
            pltpu.VMEM((chunk, _DIM), jnp.float32),
            pltpu.SemaphoreType.DMA,
            pltpu.VMEM((chunk // 128, 128), jnp.int32),
            pltpu.VMEM((chunk, _DIM), jnp.float32),
            pltpu.SemaphoreType.DMA,
        ],
        compiler_params=pltpu.CompilerParams(
            use_tc_tiling_on_sc=False, needs_layout_passes=False),
    )
    def gather(table_hbm, idx_hbm, out_hbm, table_v,
               idx_a, rows_a, sem_a, idx_b, rows_b, sem_b):
        bufs = [(idx_a, rows_a, sem_a), (idx_b, rows_b, sem_b)]
        _gather_body_loop(info, per_worker, chunk, n_chunks, groups, bufs,
                          table_hbm, idx_hbm, out_hbm, table_v)

    return gather


def kernel(X, centroids, ema_cluster_size, ema_w):
    n = X.shape[0]
    n_blk = n // _TOK_BLOCK
    parts_blk = [n_blk // 2, n_blk - n_blk // 2]
    table_t = centroids.T
    idxs, qs = [], []
    start = 0
    for pb in parts_blk:
        idx_p = _assign(X, centroids, start, pb)
        idxs.append(idx_p)
        qs.append(_make_gather(pb * _TOK_BLOCK)(table_t, idx_p))
        start += pb
    quantized = jnp.concatenate(qs, axis=0)
    idx = jnp.concatenate(idxs, axis=0).reshape(-1)[:, None]
    return (quantized, idx)
